# baseline (speedup 1.0000x reference)
import functools

import jax
import jax.numpy as jnp
from jax import lax
from jax.experimental import pallas as pl
from jax.experimental.pallas import tpu as pltpu
from jax.experimental.pallas import tpu_sc as plsc

NC = 2
NS = 16
NW = NC * NS

HALF = 256
DM = 2 * HALF
TROWS = 768
RPW = TROWS // NS
QW = 128
ECH = 64


def _body(idx_hbm, pw_hbm, ph_hbm, out_hbm, ct_hbm,
          xidx, yidx, bufw, qt, idxc, obuf, isems, ssems, sw):
    c = lax.axis_index("c")
    s = lax.axis_index("s")
    wid = s * NC + c

    r0 = s * RPW
    for j in range(3):
        t = r0 + j * 16 + lax.iota(jnp.int32, 16)
        a = t - 2
        valid = (t >= 2) & (t < 644)
        xidx[pl.ds(j * 16, 16)] = jnp.where(valid, a & 31, 32)
        yidx[pl.ds(j * 16, 16)] = jnp.where(valid, a >> 5, 20)
    ctab = ct_hbm.at[c]
    pltpu.async_copy(pw_hbm.at[xidx], bufw, sw).wait()
    pltpu.sync_copy(bufw, ctab.at[pl.ds(r0, RPW), pl.ds(0, HALF)])
    pltpu.async_copy(ph_hbm.at[yidx], bufw, sw).wait()
    pltpu.sync_copy(bufw, ctab.at[pl.ds(r0, RPW), pl.ds(HALF, HALF)])
    plsc.subcore_barrier()

    q = wid & 3
    g = wid >> 2
    n = out_hbm.shape[0]
    tpt = n // (NW // 4)
    nech = tpt // ECH
    tok0 = g * tpt
    col0 = pl.multiple_of(q * QW, QW)
    pltpu.sync_copy(ctab.at[:, pl.ds(col0, QW)], qt)

    def idx_load(j, p):
        pltpu.async_copy(idx_hbm.at[pl.ds(tok0 + j * ECH, ECH)],
                         idxc.at[pl.ds(p * ECH, ECH)], isems.at[p])

    def wait_idx(p):
        pltpu.make_async_copy(idx_hbm.at[pl.ds(tok0, ECH)],
                              idxc.at[pl.ds(p * ECH, ECH)], isems.at[p]).wait()

    def store_o(j, p):
        pltpu.async_copy(obuf.at[p],
                         out_hbm.at[pl.ds(tok0 + j * ECH, ECH),
                                    pl.ds(col0, QW)], ssems.at[p])

    def wait_store(p):
        pltpu.make_async_copy(obuf.at[p],
                              out_hbm.at[pl.ds(tok0, ECH), pl.ds(col0, QW)],
                              ssems.at[p]).wait()

    idx_load(0, 0)
    idx_load(1, 1)

    @pl.loop(0, nech, step=2)
    def _(j):
        for p in range(2):
            wait_idx(p)

            @pl.when(j + p >= 2)
            def _():
                wait_store(p)

            for w in range(ECH // 16):
                tv = idxc[pl.ds(p * ECH + w * 16, 16)]
                for z in range(16):
                    tz = tv[z]
                    for u in range(8):
                        v = qt[tz, pl.ds(u * 16, 16)]
                        obuf[p, w * 16 + z, pl.ds(u * 16, 16)] = v

            store_o(j + p, p)

            @pl.when(j + p + 2 < nech)
            def _():
                idx_load(j + p + 2, p)

    wait_store(0)
    wait_store(1)


@jax.jit
def _sc_lookup(idx_flat, pw_ext, ph_ext):
    n = idx_flat.shape[0]
    run = pl.kernel(
        _body,
        out_type=(jax.ShapeDtypeStruct((n, DM), jnp.float32),
                  jax.ShapeDtypeStruct((NC, TROWS, DM), jnp.float32)),
        mesh=plsc.VectorSubcoreMesh(core_axis_name="c", subcore_axis_name="s"),
        scratch_types=[
            pltpu.VMEM((48,), jnp.int32),
            pltpu.VMEM((48,), jnp.int32),
            pltpu.VMEM((48, HALF), jnp.float32),
            pltpu.VMEM((TROWS, QW), jnp.float32),
            pltpu.VMEM((2 * ECH,), jnp.int32),
            pltpu.VMEM((2, ECH, QW), jnp.float32),
            pltpu.SemaphoreType.DMA((2,)),
            pltpu.SemaphoreType.DMA((2,)),
            pltpu.SemaphoreType.DMA,
        ],
    )
    out, _ = run(idx_flat, pw_ext, ph_ext)
    return out


def kernel(tgt_seq, pos_w_embedding, pos_h_embedding):
    b, seq = tgt_seq.shape
    n = b * seq
    pw_ext = jnp.pad(pos_w_embedding, ((0, 2), (0, 0)))
    ph_ext = jnp.pad(pos_h_embedding, ((0, 2), (0, 0)))
    out = _sc_lookup(tgt_seq.reshape(n), pw_ext, ph_ext)
    return out.reshape(b, seq, DM)


# speedup vs baseline: 2.5530x; 2.5530x over previous
import functools

import jax
import jax.numpy as jnp
from jax import lax
from jax.experimental import pallas as pl
from jax.experimental.pallas import tpu as pltpu
from jax.experimental.pallas import tpu_sc as plsc

NC = 2
NS = 16
NW = NC * NS

HALF = 256
DM = 2 * HALF
TROWS = 768
RPW = TROWS // NS
CH = 32
NBUF = 2


def _body(idx_hbm, pw_hbm, ph_hbm, out_hbm, ct_hbm,
          xidx, yidx, bufw, idxb, obs, spb, gsems, usems, ssems, sw):
    c = lax.axis_index("c")
    s = lax.axis_index("s")
    wid = s * NC + c

    r0 = s * RPW
    for j in range(3):
        t = r0 + j * 16 + lax.iota(jnp.int32, 16)
        a = t - 2
        valid = (t >= 2) & (t < 644)
        xidx[pl.ds(j * 16, 16)] = jnp.where(valid, a & 31, 32)
        yidx[pl.ds(j * 16, 16)] = jnp.where(valid, a >> 5, 20)
    ctab = ct_hbm.at[c]
    pltpu.async_copy(pw_hbm.at[xidx], bufw, sw).wait()
    pltpu.sync_copy(bufw, ctab.at[pl.ds(r0, RPW), pl.ds(0, HALF)])
    pltpu.async_copy(ph_hbm.at[yidx], bufw, sw).wait()
    pltpu.sync_copy(bufw, ctab.at[pl.ds(r0, RPW), pl.ds(HALF, HALF)])
    plsc.subcore_barrier()

    nchunks = idxb.shape[0]
    row0 = wid * (nchunks * CH)
    pltpu.sync_copy(idx_hbm.at[wid], idxb)

    def gather(k, b):
        pltpu.async_copy(ctab.at[idxb.at[k]], obs.at[b], gsems.at[b])

    def wait_gather(b):
        pltpu.make_async_copy(ctab.at[idxb.at[0]], obs.at[b], gsems.at[b]).wait()

    def up(b):
        pltpu.async_copy(obs.at[b], spb.at[s, b], usems.at[b])

    def wait_up(b):
        pltpu.make_async_copy(obs.at[b], spb.at[s, b], usems.at[b]).wait()

    def store(k, b):
        pltpu.async_copy(spb.at[s, b], out_hbm.at[pl.ds(row0 + k * CH, CH)],
                         ssems.at[b])

    def wait_store(b):
        pltpu.make_async_copy(spb.at[s, b], out_hbm.at[pl.ds(row0, CH)],
                              ssems.at[b]).wait()

    for b in range(NBUF):
        gather(b, b)

    @pl.loop(0, nchunks, step=NBUF)
    def _(k):
        for b in range(NBUF):

            @pl.when(k + b < nchunks)
            def _():
                wait_gather(b)

                @pl.when(k + b >= NBUF)
                def _():
                    wait_store(b)
                up(b)
        for b in range(NBUF):

            @pl.when(k + b < nchunks)
            def _():
                wait_up(b)
                store(k + b, b)
                nxt = k + b + NBUF

                @pl.when(nxt < nchunks)
                def _():
                    gather(nxt, b)

    for b in range(NBUF):
        wait_store(b)


@jax.jit
def _sc_lookup(idx3d, pw_ext, ph_ext):
    n = idx3d.shape[0] * idx3d.shape[1] * idx3d.shape[2]
    nchunks = n // (NW * CH)
    run = pl.kernel(
        _body,
        out_type=(jax.ShapeDtypeStruct((n, DM), jnp.float32),
                  jax.ShapeDtypeStruct((NC, TROWS, DM), jnp.float32)),
        mesh=plsc.VectorSubcoreMesh(core_axis_name="c", subcore_axis_name="s"),
        scratch_types=[
            pltpu.VMEM((48,), jnp.int32),
            pltpu.VMEM((48,), jnp.int32),
            pltpu.VMEM((48, HALF), jnp.float32),
            pltpu.VMEM((nchunks, CH), jnp.int32),
            pltpu.VMEM((NBUF, CH, DM), jnp.float32),
            pltpu.VMEM_SHARED((NS, NBUF, CH, DM), jnp.float32),
            pltpu.SemaphoreType.DMA((NBUF,)),
            pltpu.SemaphoreType.DMA((NBUF,)),
            pltpu.SemaphoreType.DMA((NBUF,)),
            pltpu.SemaphoreType.DMA,
        ],
    )
    out, _ = run(idx3d, pw_ext, ph_ext)
    return out


def kernel(tgt_seq, pos_w_embedding, pos_h_embedding):
    b, seq = tgt_seq.shape
    n = b * seq
    pw_ext = jnp.pad(pos_w_embedding, ((0, 2), (0, 0)))
    ph_ext = jnp.pad(pos_h_embedding, ((0, 2), (0, 0)))
    idx3d = tgt_seq.reshape(NW, n // (NW * CH), CH)
    out = _sc_lookup(idx3d, pw_ext, ph_ext)
    return out.reshape(b, seq, DM)


# speedup vs baseline: 2.6031x; 1.0196x over previous
import functools

import jax
import jax.numpy as jnp
from jax import lax
from jax.experimental import pallas as pl
from jax.experimental.pallas import tpu as pltpu
from jax.experimental.pallas import tpu_sc as plsc

NC = 2
NS = 16
NW = NC * NS

HALF = 256
DM = 2 * HALF
TROWS = 768
RPW = TROWS // NS
CH = 64
NBUF = 2


def _body(idx_hbm, pw_hbm, ph_hbm, out_hbm, ct_hbm,
          xidx, yidx, bufw, idxb, obs, gsems, ssems, sw):
    c = lax.axis_index("c")
    s = lax.axis_index("s")
    wid = s * NC + c

    r0 = s * RPW
    for j in range(3):
        t = r0 + j * 16 + lax.iota(jnp.int32, 16)
        a = t - 2
        valid = (t >= 2) & (t < 644)
        xidx[pl.ds(j * 16, 16)] = jnp.where(valid, a & 31, 32)
        yidx[pl.ds(j * 16, 16)] = jnp.where(valid, a >> 5, 20)
    ctab = ct_hbm.at[c]
    pltpu.async_copy(pw_hbm.at[xidx], bufw, sw).wait()
    pltpu.sync_copy(bufw, ctab.at[pl.ds(r0, RPW), pl.ds(0, HALF)])
    pltpu.async_copy(ph_hbm.at[yidx], bufw, sw).wait()
    pltpu.sync_copy(bufw, ctab.at[pl.ds(r0, RPW), pl.ds(HALF, HALF)])
    plsc.subcore_barrier()

    nchunks = idxb.shape[0]
    row0 = wid * (nchunks * CH)
    pltpu.sync_copy(idx_hbm.at[wid], idxb)

    def gather(k, b):
        pltpu.async_copy(ctab.at[idxb.at[k]], obs.at[b], gsems.at[b])

    def wait_gather(b):
        pltpu.make_async_copy(ctab.at[idxb.at[0]], obs.at[b], gsems.at[b]).wait()

    def store(k, b):
        pltpu.async_copy(obs.at[b], out_hbm.at[pl.ds(row0 + k * CH, CH)],
                         ssems.at[b])

    def wait_store(b):
        pltpu.make_async_copy(obs.at[b], out_hbm.at[pl.ds(row0, CH)],
                              ssems.at[b]).wait()

    for b in range(NBUF):
        gather(b, b)

    @pl.loop(0, nchunks, step=NBUF)
    def _(k):
        for b in range(NBUF):

            @pl.when(k + b < nchunks)
            def _():
                wait_gather(b)
                store(k + b, b)
        for b in range(NBUF):
            nxt = k + b + NBUF

            @pl.when(nxt < nchunks)
            def _():
                wait_store(b)
                gather(nxt, b)

    for b in range(NBUF):
        wait_store(b)


@jax.jit
def _sc_lookup(idx3d, pw_ext, ph_ext):
    n = idx3d.shape[0] * idx3d.shape[1] * idx3d.shape[2]
    nchunks = n // (NW * CH)
    run = pl.kernel(
        _body,
        out_type=(jax.ShapeDtypeStruct((n, DM), jnp.float32),
                  jax.ShapeDtypeStruct((NC, TROWS, DM), jnp.float32)),
        mesh=plsc.VectorSubcoreMesh(core_axis_name="c", subcore_axis_name="s"),
        scratch_types=[
            pltpu.VMEM((48,), jnp.int32),
            pltpu.VMEM((48,), jnp.int32),
            pltpu.VMEM((48, HALF), jnp.float32),
            pltpu.VMEM((nchunks, CH), jnp.int32),
            pltpu.VMEM((NBUF, CH, DM), jnp.float32),
            pltpu.SemaphoreType.DMA((NBUF,)),
            pltpu.SemaphoreType.DMA((NBUF,)),
            pltpu.SemaphoreType.DMA,
        ],
    )
    out, _ = run(idx3d, pw_ext, ph_ext)
    return out


def kernel(tgt_seq, pos_w_embedding, pos_h_embedding):
    b, seq = tgt_seq.shape
    n = b * seq
    pw_ext = jnp.pad(pos_w_embedding, ((0, 2), (0, 0)))
    ph_ext = jnp.pad(pos_h_embedding, ((0, 2), (0, 0)))
    idx3d = tgt_seq.reshape(NW, n // (NW * CH), CH)
    out = _sc_lookup(idx3d, pw_ext, ph_ext)
    return out.reshape(b, seq, DM)
